# Initial kernel scaffold; baseline (speedup 1.0000x reference)
import jax
import jax.numpy as jnp
from jax.experimental import pallas as pl


def kernel(patches, pos_table):
    raise NotImplementedError("write your pallas kernel here")


# speedup vs baseline: 1.0125x; 1.0125x over previous
import jax
import jax.numpy as jnp
from jax.experimental import pallas as pl
from jax.experimental.pallas import tpu as pltpu

NUM_PATCHES = 1024
PROJ_DIM = 768
BATCH = 64


def _add_body(patches_ref, pos_ref, out_ref):
    out_ref[...] = patches_ref[...] + pos_ref[...][None]


def kernel(patches, pos_table):
    grid = (BATCH,)
    return pl.pallas_call(
        _add_body,
        grid=grid,
        in_specs=[
            pl.BlockSpec((1, NUM_PATCHES, PROJ_DIM), lambda b: (b, 0, 0)),
            pl.BlockSpec((NUM_PATCHES, PROJ_DIM), lambda b: (0, 0)),
        ],
        out_specs=pl.BlockSpec((1, NUM_PATCHES, PROJ_DIM), lambda b: (b, 0, 0)),
        out_shape=jax.ShapeDtypeStruct((BATCH, NUM_PATCHES, PROJ_DIM), jnp.float32),
    )(patches, pos_table)
